# Initial kernel scaffold; baseline (speedup 1.0000x reference)
import jax
import jax.numpy as jnp
from jax.experimental import pallas as pl


def kernel(batch, W, b):
    raise NotImplementedError("write your pallas kernel here")


# speedup vs baseline: 4.0546x; 4.0546x over previous
import functools

import jax
import jax.numpy as jnp
from jax import lax
from jax.experimental import pallas as pl
from jax.experimental.pallas import tpu as pltpu
from jax.experimental.pallas import tpu_sc as plsc

_B, _L, _D, _F = 16, 2048, 128, 128
_NC, _NS, _LANES = 2, 16, 16
_NW = _NC * _NS
_ROWS_PER_W = (_B * _L) // _NW
_CHUNK = 256
_NCHUNK = _ROWS_PER_W // _CHUNK
_NVEC = _D // _LANES


def _sc_reduce_body(x_hbm, out_hbm, buf, stage):
    c = lax.axis_index("c")
    s = lax.axis_index("s")
    wid = s * _NC + c
    set_id = wid % _B
    half = wid // _B
    base = set_id * _L + half * _ROWS_PER_W

    accs = tuple(jnp.zeros((_LANES,), jnp.float32) for _ in range(_NVEC))
    for ck in range(_NCHUNK):
        pltpu.sync_copy(x_hbm.at[pl.ds(base + ck * _CHUNK, _CHUNK)], buf)

        def body(r, a):
            return tuple(a[f] + buf[r, pl.ds(f * _LANES, _LANES)]
                         for f in range(_NVEC))

        accs = lax.fori_loop(0, _CHUNK, body, accs)

    for f in range(_NVEC):
        stage[0, pl.ds(f * _LANES, _LANES)] = accs[f]
    pltpu.sync_copy(stage, out_hbm.at[pl.ds(wid, 1)])


_sc_reduce = functools.partial(
    pl.kernel,
    mesh=plsc.VectorSubcoreMesh(core_axis_name="c", subcore_axis_name="s"),
    out_type=jax.ShapeDtypeStruct((_NW, _D), jnp.float32),
    scratch_types=[
        pltpu.VMEM((_CHUNK, _D), jnp.float32),
        pltpu.VMEM((1, _D), jnp.float32),
    ],
)(_sc_reduce_body)


def _tc_affine_body(p_ref, w_ref, b_ref, o_ref):
    s = p_ref[0:_B, :] + p_ref[_B:2 * _B, :]
    o_ref[...] = (
        lax.dot_general(s, w_ref[...], (((1,), (1,)), ((), ())),
                        preferred_element_type=jnp.float32)
        + jnp.float32(_L) * b_ref[...]
    )


def kernel(batch, W, b):
    x = batch.reshape(_B * _L, _D)
    partial = _sc_reduce(x)
    out = pl.pallas_call(
        _tc_affine_body,
        out_shape=jax.ShapeDtypeStruct((_B, _F), jnp.float32),
    )(partial, W, b.reshape(1, _F))
    return out
